# Initial kernel scaffold; baseline (speedup 1.0000x reference)
import jax
import jax.numpy as jnp
from jax.experimental import pallas as pl


def kernel(pred_logits, pred_keypoints, tgt_keypoints, tgt_ids):
    raise NotImplementedError("write your pallas kernel here")


# speedup vs baseline: 1.5415x; 1.5415x over previous
import jax
import jax.numpy as jnp
from jax.experimental import pallas as pl
from jax.experimental.pallas import tpu as pltpu

_L_DELTAS = 0.5
_L_VIS = 0.2
_L_CTR = 0.5
_L_ABS = 4.0
_COST_CLASS = 1.0
_EPS = 1e-12


def _cost_body(logits_ref, cp_ref, zp_ref, vp_ref,
               cgT_ref, zgT_ref, vgT_ref, ids_ref, out_ref):
    logits = logits_ref[0]
    cp = cp_ref[0]
    zp = zp_ref[0]
    vp = vp_ref[0]
    cgT = cgT_ref[...]
    zgT = zgT_ref[...]
    vgT = vgT_ref[...]
    ids = ids_ref[...]

    l0 = logits[:, 0:1]
    l1 = logits[:, 1:2]
    m = jnp.maximum(l0, l1)
    e0 = jnp.exp(l0 - m)
    e1 = jnp.exp(l1 - m)
    inv = 1.0 / (e0 + e1)
    p0 = e0 * inv
    p1 = e1 * inv
    is0 = (ids == 0).astype(jnp.float32)
    cost_class = -(p1 + (p0 - p1) * is0)

    dcx = cp[:, 0:1] - cgT[0:1, :]
    dcy = cp[:, 1:2] - cgT[1:2, :]
    center = jnp.sqrt(jnp.maximum(dcx * dcx + dcy * dcy, _EPS))

    vacc = jnp.zeros_like(dcx)
    for j in range(17):
        r = vp[:, j:j + 1] - vgT[j:j + 1, :]
        vacc = vacc + r * r
    viz = jnp.sqrt(jnp.maximum(vacc, _EPS))

    acc = (_COST_CLASS * cost_class + _L_VIS * viz + _L_CTR * center)

    w_off = _L_DELTAS * jnp.abs(vgT)
    w_abs = _L_ABS * jnp.abs(vgT)
    for d in range(34):
        j = d // 2
        u = zp[:, d:d + 1] - zgT[d:d + 1, :]
        a = u + (dcx if d % 2 == 0 else dcy)
        acc = acc + jnp.abs(u) * w_off[j:j + 1, :]
        acc = acc + jnp.abs(a) * w_abs[j:j + 1, :]

    out_ref[0] = acc


@jax.jit
def kernel(pred_logits, pred_keypoints, tgt_keypoints, tgt_ids):
    bs, nq, ncls = pred_logits.shape
    T = tgt_keypoints.shape[0]

    cp = pred_keypoints[..., 0:2]
    zp = pred_keypoints[..., 2:36]
    vp = pred_keypoints[..., 36:53]
    tgtT = tgt_keypoints.T
    cgT = tgtT[0:2]
    zgT = tgtT[2:36]
    vgT = tgtT[36:53]
    ids2d = tgt_ids.reshape(1, T).astype(jnp.int32)

    return pl.pallas_call(
        _cost_body,
        grid=(bs,),
        in_specs=[
            pl.BlockSpec((1, nq, ncls), lambda b: (b, 0, 0)),
            pl.BlockSpec((1, nq, 2), lambda b: (b, 0, 0)),
            pl.BlockSpec((1, nq, 34), lambda b: (b, 0, 0)),
            pl.BlockSpec((1, nq, 17), lambda b: (b, 0, 0)),
            pl.BlockSpec((2, T), lambda b: (0, 0)),
            pl.BlockSpec((34, T), lambda b: (0, 0)),
            pl.BlockSpec((17, T), lambda b: (0, 0)),
            pl.BlockSpec((1, T), lambda b: (0, 0)),
        ],
        out_specs=pl.BlockSpec((1, nq, T), lambda b: (b, 0, 0)),
        out_shape=jax.ShapeDtypeStruct((bs, nq, T), jnp.float32),
        compiler_params=pltpu.CompilerParams(
            dimension_semantics=("parallel",),
        ),
    )(pred_logits, cp, zp, vp, cgT, zgT, vgT, ids2d)


# speedup vs baseline: 2.1185x; 1.3743x over previous
import jax
import jax.numpy as jnp
from jax.experimental import pallas as pl
from jax.experimental.pallas import tpu as pltpu

_L_DELTAS = 0.5
_L_VIS = 0.2
_L_CTR = 0.5
_L_ABS = 4.0
_COST_CLASS = 1.0
_EPS = 1e-12
_NQ_BLK = 128
_NACC = 8

_BF = jnp.bfloat16
_F32 = jnp.float32


def _cost_body(logits_ref, cp_ref, zpb_ref, vpb_ref,
               cgT_ref, zgR_ref, vgR_ref, w05R_ref, w4R_ref, ids_ref,
               out_ref):
    nq = out_ref.shape[1]
    T = out_ref.shape[2]
    rep = nq // 16

    logits = logits_ref[0]
    cp = cp_ref[0]
    zpb = zpb_ref[0]
    vpb = vpb_ref[0]
    cgT = cgT_ref[...]
    ids = ids_ref[...]

    l0 = logits[:, 0:1]
    l1 = logits[:, 1:2]
    m = jnp.maximum(l0, l1)
    e0 = jnp.exp(l0 - m)
    e1 = jnp.exp(l1 - m)
    inv = 1.0 / (e0 + e1)
    p0 = e0 * inv
    p1 = e1 * inv
    is0 = (ids == 0).astype(_F32)
    cost_class = -(p1 + (p0 - p1) * is0)

    dcx = cp[:, 0:1] - cgT[0:1, :]
    dcy = cp[:, 1:2] - cgT[1:2, :]
    center = jnp.sqrt(jnp.maximum(dcx * dcx + dcy * dcy, _EPS))
    dcxb = dcx.astype(_BF)
    dcyb = dcy.astype(_BF)

    vaccs = [jnp.zeros((nq, T), _BF) for _ in range(2)]
    for j in range(17):
        row = pltpu.repeat(vgR_ref[j], rep, 0)
        r = vpb[:, j:j + 1] - row
        vaccs[j % 2] = vaccs[j % 2] + r * r
    viz = jnp.sqrt(jnp.maximum(
        vaccs[0].astype(_F32) + vaccs[1].astype(_F32), _EPS))

    accs = [jnp.zeros((nq, T), _BF) for _ in range(_NACC)]
    for d in range(34):
        j = d // 2
        zrow = pltpu.repeat(zgR_ref[d], rep, 0)
        w05 = pltpu.repeat(w05R_ref[j], rep, 0)
        w4 = pltpu.repeat(w4R_ref[j], rep, 0)
        u = zpb[:, d:d + 1] - zrow
        a = u + (dcxb if d % 2 == 0 else dcyb)
        t = jnp.abs(u) * w05 + jnp.abs(a) * w4
        accs[d % _NACC] = accs[d % _NACC] + t

    acc = accs[0].astype(_F32)
    for g in range(1, _NACC):
        acc = acc + accs[g].astype(_F32)

    out_ref[0] = (_COST_CLASS * cost_class + _L_VIS * viz
                  + _L_CTR * center + acc)


@jax.jit
def kernel(pred_logits, pred_keypoints, tgt_keypoints, tgt_ids):
    bs, nq, ncls = pred_logits.shape
    T = tgt_keypoints.shape[0]

    cp = pred_keypoints[..., 0:2]
    zpb = pred_keypoints[..., 2:36].astype(_BF)
    vpb = pred_keypoints[..., 36:53].astype(_BF)

    tgtT = tgt_keypoints.T
    cgT = tgtT[0:2]
    zgR = jnp.broadcast_to(
        tgtT[2:36].astype(_BF)[:, None, :], (34, 16, T))
    vgR = jnp.broadcast_to(
        tgtT[36:53].astype(_BF)[:, None, :], (17, 16, T))
    wabs = jnp.abs(tgtT[36:53])
    w05R = jnp.broadcast_to(
        (_L_DELTAS * wabs).astype(_BF)[:, None, :], (17, 16, T))
    w4R = jnp.broadcast_to(
        (_L_ABS * wabs).astype(_BF)[:, None, :], (17, 16, T))
    ids2d = tgt_ids.reshape(1, T).astype(jnp.int32)

    nblk = nq // _NQ_BLK
    return pl.pallas_call(
        _cost_body,
        grid=(bs, nblk),
        in_specs=[
            pl.BlockSpec((1, _NQ_BLK, ncls), lambda b, q: (b, q, 0)),
            pl.BlockSpec((1, _NQ_BLK, 2), lambda b, q: (b, q, 0)),
            pl.BlockSpec((1, _NQ_BLK, 34), lambda b, q: (b, q, 0)),
            pl.BlockSpec((1, _NQ_BLK, 17), lambda b, q: (b, q, 0)),
            pl.BlockSpec((2, T), lambda b, q: (0, 0)),
            pl.BlockSpec((34, 16, T), lambda b, q: (0, 0, 0)),
            pl.BlockSpec((17, 16, T), lambda b, q: (0, 0, 0)),
            pl.BlockSpec((17, 16, T), lambda b, q: (0, 0, 0)),
            pl.BlockSpec((17, 16, T), lambda b, q: (0, 0, 0)),
            pl.BlockSpec((1, T), lambda b, q: (0, 0)),
        ],
        out_specs=pl.BlockSpec((1, _NQ_BLK, T), lambda b, q: (b, q, 0)),
        out_shape=jax.ShapeDtypeStruct((bs, nq, T), jnp.float32),
        compiler_params=pltpu.CompilerParams(
            dimension_semantics=("parallel", "arbitrary"),
        ),
    )(pred_logits, cp, zpb, vpb, cgT, zgR, vgR, w05R, w4R, ids2d)


# speedup vs baseline: 2.4405x; 1.1520x over previous
import jax
import jax.numpy as jnp
from jax.experimental import pallas as pl
from jax.experimental.pallas import tpu as pltpu

_L_DELTAS = 0.5
_L_VIS = 0.2
_L_CTR = 0.5
_L_ABS = 4.0
_COST_CLASS = 1.0
_EPS = 1e-12
_NQ_BLK = 128
_NACC = 8

_BF = jnp.bfloat16
_F32 = jnp.float32


def _cost_body(logits_ref, cp_ref, zpb_ref, vpb_ref,
               cgT_ref, zgR_ref, w05R_ref, w4R_ref, vgb_ref, ids_ref,
               out_ref):
    nq = out_ref.shape[1]
    T = out_ref.shape[2]
    rep = nq // 16

    logits = logits_ref[0]
    cp = cp_ref[0]
    zpb = zpb_ref[0]
    vpb = vpb_ref[0]
    cgT = cgT_ref[...]
    vgb = vgb_ref[...]
    ids = ids_ref[...]

    l0 = logits[:, 0:1]
    l1 = logits[:, 1:2]
    m = jnp.maximum(l0, l1)
    e0 = jnp.exp(l0 - m)
    e1 = jnp.exp(l1 - m)
    inv = 1.0 / (e0 + e1)
    p0 = e0 * inv
    p1 = e1 * inv
    is0 = (ids == 0).astype(_F32)
    cost_class = -(p1 + (p0 - p1) * is0)

    dcx = cp[:, 0:1] - cgT[0:1, :]
    dcy = cp[:, 1:2] - cgT[1:2, :]
    center = jnp.sqrt(jnp.maximum(dcx * dcx + dcy * dcy, _EPS))
    dcxb = dcx.astype(_BF)
    dcyb = dcy.astype(_BF)

    vdot = jax.lax.dot_general(vpb, vgb, (((1,), (0,)), ((), ())),
                               preferred_element_type=_F32)
    vpf = vpb.astype(_F32)
    npred = jnp.sum(vpf * vpf, axis=1, keepdims=True)
    vgf = vgb.astype(_F32)
    ntgt = jnp.sum(vgf * vgf, axis=0, keepdims=True)
    viz = jnp.sqrt(jnp.maximum(npred + ntgt - 2.0 * vdot, _EPS))

    base = cost_class + _L_VIS * viz + _L_CTR * center

    accs = [jnp.zeros((nq, T), _BF) for _ in range(_NACC)]
    for j in range(17):
        w05 = pltpu.repeat(w05R_ref[j], rep, 0)
        w4 = pltpu.repeat(w4R_ref[j], rep, 0)
        for k in (0, 1):
            d = 2 * j + k
            zrow = pltpu.repeat(zgR_ref[d], rep, 0)
            u = zpb[:, d:d + 1] - zrow
            a = u + (dcxb if k == 0 else dcyb)
            t = jnp.abs(u) * w05 + jnp.abs(a) * w4
            accs[d % _NACC] = accs[d % _NACC] + t

    acc = base
    for g in range(_NACC):
        acc = acc + accs[g].astype(_F32)

    out_ref[0] = acc


@jax.jit
def kernel(pred_logits, pred_keypoints, tgt_keypoints, tgt_ids):
    bs, nq, ncls = pred_logits.shape
    T = tgt_keypoints.shape[0]

    cp = pred_keypoints[..., 0:2]
    zpb = pred_keypoints[..., 2:36].astype(_BF)
    vpb = pred_keypoints[..., 36:53].astype(_BF)

    tgtT = tgt_keypoints.T
    cgT = tgtT[0:2]
    zgR = jnp.broadcast_to(
        tgtT[2:36].astype(_BF)[:, None, :], (34, 16, T))
    vgb = tgtT[36:53].astype(_BF)
    wabs = jnp.abs(tgtT[36:53])
    w05R = jnp.broadcast_to(
        (_L_DELTAS * wabs).astype(_BF)[:, None, :], (17, 16, T))
    w4R = jnp.broadcast_to(
        (_L_ABS * wabs).astype(_BF)[:, None, :], (17, 16, T))
    ids2d = tgt_ids.reshape(1, T).astype(jnp.int32)

    nblk = nq // _NQ_BLK
    return pl.pallas_call(
        _cost_body,
        grid=(bs, nblk),
        in_specs=[
            pl.BlockSpec((1, _NQ_BLK, ncls), lambda b, q: (b, q, 0)),
            pl.BlockSpec((1, _NQ_BLK, 2), lambda b, q: (b, q, 0)),
            pl.BlockSpec((1, _NQ_BLK, 34), lambda b, q: (b, q, 0)),
            pl.BlockSpec((1, _NQ_BLK, 17), lambda b, q: (b, q, 0)),
            pl.BlockSpec((2, T), lambda b, q: (0, 0)),
            pl.BlockSpec((34, 16, T), lambda b, q: (0, 0, 0)),
            pl.BlockSpec((17, 16, T), lambda b, q: (0, 0, 0)),
            pl.BlockSpec((17, 16, T), lambda b, q: (0, 0, 0)),
            pl.BlockSpec((17, T), lambda b, q: (0, 0)),
            pl.BlockSpec((1, T), lambda b, q: (0, 0)),
        ],
        out_specs=pl.BlockSpec((1, _NQ_BLK, T), lambda b, q: (b, q, 0)),
        out_shape=jax.ShapeDtypeStruct((bs, nq, T), jnp.float32),
        compiler_params=pltpu.CompilerParams(
            dimension_semantics=("parallel", "arbitrary"),
        ),
    )(pred_logits, cp, zpb, vpb, cgT, zgR, w05R, w4R, vgb, ids2d)


# speedup vs baseline: 2.5563x; 1.0474x over previous
import jax
import jax.numpy as jnp
from jax.experimental import pallas as pl
from jax.experimental.pallas import tpu as pltpu

_L_DELTAS = 0.5
_L_VIS = 0.2
_L_CTR = 0.5
_L_ABS = 4.0
_COST_CLASS = 1.0
_EPS = 1e-12
_NQ_BLK = 128
_GROUP = 6

_BF = jnp.bfloat16
_F32 = jnp.float32


def _cost_body(logits_ref, kp_ref,
               cgT_ref, zgR_ref, w05R_ref, w4R_ref, vgb_ref, ids_ref,
               out_ref):
    nq = out_ref.shape[1]
    T = out_ref.shape[2]
    rep = nq // 16

    logits = logits_ref[0]
    kp = kp_ref[0]
    cp = kp[:, 0:2]
    zpb = kp[:, 2:36].astype(_BF)
    vpb = kp[:, 36:53].astype(_BF)
    cgT = cgT_ref[...]
    vgb = vgb_ref[...]
    ids = ids_ref[...]

    l0 = logits[:, 0:1]
    l1 = logits[:, 1:2]
    m = jnp.maximum(l0, l1)
    e0 = jnp.exp(l0 - m)
    e1 = jnp.exp(l1 - m)
    inv = 1.0 / (e0 + e1)
    p0 = e0 * inv
    p1 = e1 * inv
    is0 = (ids == 0).astype(_F32)
    cost_class = -(p1 + (p0 - p1) * is0)

    dcx = cp[:, 0:1] - cgT[0:1, :]
    dcy = cp[:, 1:2] - cgT[1:2, :]
    center = jnp.sqrt(jnp.maximum(dcx * dcx + dcy * dcy, _EPS))
    dcxb = dcx.astype(_BF)
    dcyb = dcy.astype(_BF)

    vdot = jax.lax.dot_general(vpb, vgb, (((1,), (0,)), ((), ())),
                               preferred_element_type=_F32)
    vpf = vpb.astype(_F32)
    npred = jnp.sum(vpf * vpf, axis=1, keepdims=True)
    vgf = vgb.astype(_F32)
    ntgt = jnp.sum(vgf * vgf, axis=0, keepdims=True)
    viz = jnp.sqrt(jnp.maximum(npred + ntgt - 2.0 * vdot, _EPS))

    base = cost_class + _L_VIS * viz + _L_CTR * center

    acc = base
    group = None
    for j in range(17):
        w05 = pltpu.repeat(w05R_ref[j], rep, 0)
        w4 = pltpu.repeat(w4R_ref[j], rep, 0)
        for k in (0, 1):
            d = 2 * j + k
            zrow = pltpu.repeat(zgR_ref[d], rep, 0)
            u = zpb[:, d:d + 1] - zrow
            a = u + (dcxb if k == 0 else dcyb)
            t = jnp.abs(u) * w05 + jnp.abs(a) * w4
            group = t if group is None else group + t
            if d % _GROUP == _GROUP - 1 or d == 33:
                acc = acc + group.astype(_F32)
                group = None

    out_ref[0] = acc


@jax.jit
def kernel(pred_logits, pred_keypoints, tgt_keypoints, tgt_ids):
    bs, nq, ncls = pred_logits.shape
    T = tgt_keypoints.shape[0]

    tgtT = tgt_keypoints.T
    cgT = tgtT[0:2]
    zgR = jnp.broadcast_to(
        tgtT[2:36].astype(_BF)[:, None, :], (34, 16, T))
    vgb = tgtT[36:53].astype(_BF)
    wabs = jnp.abs(tgtT[36:53])
    w05R = jnp.broadcast_to(
        (_L_DELTAS * wabs).astype(_BF)[:, None, :], (17, 16, T))
    w4R = jnp.broadcast_to(
        (_L_ABS * wabs).astype(_BF)[:, None, :], (17, 16, T))
    ids2d = tgt_ids.reshape(1, T).astype(jnp.int32)

    nblk = nq // _NQ_BLK
    return pl.pallas_call(
        _cost_body,
        grid=(bs, nblk),
        in_specs=[
            pl.BlockSpec((1, _NQ_BLK, ncls), lambda b, q: (b, q, 0)),
            pl.BlockSpec((1, _NQ_BLK, 53), lambda b, q: (b, q, 0)),
            pl.BlockSpec((2, T), lambda b, q: (0, 0)),
            pl.BlockSpec((34, 16, T), lambda b, q: (0, 0, 0)),
            pl.BlockSpec((17, 16, T), lambda b, q: (0, 0, 0)),
            pl.BlockSpec((17, 16, T), lambda b, q: (0, 0, 0)),
            pl.BlockSpec((17, T), lambda b, q: (0, 0)),
            pl.BlockSpec((1, T), lambda b, q: (0, 0)),
        ],
        out_specs=pl.BlockSpec((1, _NQ_BLK, T), lambda b, q: (b, q, 0)),
        out_shape=jax.ShapeDtypeStruct((bs, nq, T), jnp.float32),
        compiler_params=pltpu.CompilerParams(
            dimension_semantics=("parallel", "arbitrary"),
        ),
    )(pred_logits, pred_keypoints, cgT, zgR, w05R, w4R, vgb, ids2d)


# speedup vs baseline: 2.7207x; 1.0643x over previous
import jax
import jax.numpy as jnp
from jax.experimental import pallas as pl
from jax.experimental.pallas import tpu as pltpu

_L_DELTAS = 0.5
_L_VIS = 0.2
_L_CTR = 0.5
_L_ABS = 4.0
_COST_CLASS = 1.0
_EPS = 1e-12
_NQ_BLK = 256
_GROUP = 6

_BF = jnp.bfloat16
_F32 = jnp.float32


def _cost_body(logits_ref, kp_ref,
               cgT_ref, zgR_ref, w05R_ref, w4R_ref, vgb_ref, ids_ref,
               out_ref):
    nq = out_ref.shape[1]
    T = out_ref.shape[2]
    rep = nq // 16

    logits = logits_ref[0]
    kp = kp_ref[0]
    cp = kp[:, 0:2]
    zpb = kp[:, 2:36].astype(_BF)
    vpb = kp[:, 36:53].astype(_BF)
    cgT = cgT_ref[...]
    vgb = vgb_ref[...]
    ids = ids_ref[...]

    l0 = logits[:, 0:1]
    l1 = logits[:, 1:2]
    m = jnp.maximum(l0, l1)
    e0 = jnp.exp(l0 - m)
    e1 = jnp.exp(l1 - m)
    inv = 1.0 / (e0 + e1)
    p0 = e0 * inv
    p1 = e1 * inv
    is0 = (ids == 0).astype(_F32)
    cost_class = -(p1 + (p0 - p1) * is0)

    dcx = cp[:, 0:1] - cgT[0:1, :]
    dcy = cp[:, 1:2] - cgT[1:2, :]
    center = jnp.sqrt(jnp.maximum(dcx * dcx + dcy * dcy, _EPS))
    dcxb = dcx.astype(_BF)
    dcyb = dcy.astype(_BF)

    vdot = jax.lax.dot_general(vpb, vgb, (((1,), (0,)), ((), ())),
                               preferred_element_type=_F32)
    vpf = vpb.astype(_F32)
    npred = jnp.sum(vpf * vpf, axis=1, keepdims=True)
    vgf = vgb.astype(_F32)
    ntgt = jnp.sum(vgf * vgf, axis=0, keepdims=True)
    viz = jnp.sqrt(jnp.maximum(npred + ntgt - 2.0 * vdot, _EPS))

    base = cost_class + _L_VIS * viz + _L_CTR * center

    acc = base
    group = None
    for j in range(17):
        w05 = pltpu.repeat(w05R_ref[j], rep, 0)
        w4 = pltpu.repeat(w4R_ref[j], rep, 0)
        for k in (0, 1):
            d = 2 * j + k
            zrow = pltpu.repeat(zgR_ref[d], rep, 0)
            u = zpb[:, d:d + 1] - zrow
            a = u + (dcxb if k == 0 else dcyb)
            t = jnp.abs(u) * w05 + jnp.abs(a) * w4
            group = t if group is None else group + t
            if d % _GROUP == _GROUP - 1 or d == 33:
                acc = acc + group.astype(_F32)
                group = None

    out_ref[0] = acc


@jax.jit
def kernel(pred_logits, pred_keypoints, tgt_keypoints, tgt_ids):
    bs, nq, ncls = pred_logits.shape
    T = tgt_keypoints.shape[0]

    tgtT = tgt_keypoints.T
    cgT = tgtT[0:2]
    zgR = jnp.broadcast_to(
        tgtT[2:36].astype(_BF)[:, None, :], (34, 16, T))
    vgb = tgtT[36:53].astype(_BF)
    wabs = jnp.abs(tgtT[36:53])
    w05R = jnp.broadcast_to(
        (_L_DELTAS * wabs).astype(_BF)[:, None, :], (17, 16, T))
    w4R = jnp.broadcast_to(
        (_L_ABS * wabs).astype(_BF)[:, None, :], (17, 16, T))
    ids2d = tgt_ids.reshape(1, T).astype(jnp.int32)

    nblk = nq // _NQ_BLK
    return pl.pallas_call(
        _cost_body,
        grid=(bs, nblk),
        in_specs=[
            pl.BlockSpec((1, _NQ_BLK, ncls), lambda b, q: (b, q, 0)),
            pl.BlockSpec((1, _NQ_BLK, 53), lambda b, q: (b, q, 0)),
            pl.BlockSpec((2, T), lambda b, q: (0, 0)),
            pl.BlockSpec((34, 16, T), lambda b, q: (0, 0, 0)),
            pl.BlockSpec((17, 16, T), lambda b, q: (0, 0, 0)),
            pl.BlockSpec((17, 16, T), lambda b, q: (0, 0, 0)),
            pl.BlockSpec((17, T), lambda b, q: (0, 0)),
            pl.BlockSpec((1, T), lambda b, q: (0, 0)),
        ],
        out_specs=pl.BlockSpec((1, _NQ_BLK, T), lambda b, q: (b, q, 0)),
        out_shape=jax.ShapeDtypeStruct((bs, nq, T), jnp.float32),
        compiler_params=pltpu.CompilerParams(
            dimension_semantics=("parallel", "arbitrary"),
        ),
    )(pred_logits, pred_keypoints, cgT, zgR, w05R, w4R, vgb, ids2d)


# speedup vs baseline: 2.7695x; 1.0180x over previous
import jax
import jax.numpy as jnp
from jax.experimental import pallas as pl
from jax.experimental.pallas import tpu as pltpu

_L_DELTAS = 0.5
_L_VIS = 0.2
_L_CTR = 0.5
_L_ABS = 4.0
_COST_CLASS = 1.0
_EPS = 1e-12
_NQ_BLK = 512
_GROUP = 6

_BF = jnp.bfloat16
_F32 = jnp.float32


def _cost_body(logits_ref, kp_ref,
               cgT_ref, zgR_ref, w05R_ref, w4R_ref, vgb_ref, ids_ref,
               out_ref):
    nq = out_ref.shape[1]
    T = out_ref.shape[2]
    rep = nq // 16

    logits = logits_ref[0]
    kp = kp_ref[0]
    cp = kp[:, 0:2]
    zpb = kp[:, 2:36].astype(_BF)
    vpb = kp[:, 36:53].astype(_BF)
    cgT = cgT_ref[...]
    vgb = vgb_ref[...]
    ids = ids_ref[...]

    l0 = logits[:, 0:1]
    l1 = logits[:, 1:2]
    m = jnp.maximum(l0, l1)
    e0 = jnp.exp(l0 - m)
    e1 = jnp.exp(l1 - m)
    inv = 1.0 / (e0 + e1)
    p0 = e0 * inv
    p1 = e1 * inv
    is0 = (ids == 0).astype(_F32)
    cost_class = -(p1 + (p0 - p1) * is0)

    dcx = cp[:, 0:1] - cgT[0:1, :]
    dcy = cp[:, 1:2] - cgT[1:2, :]
    center = jnp.sqrt(jnp.maximum(dcx * dcx + dcy * dcy, _EPS))
    dcxb = dcx.astype(_BF)
    dcyb = dcy.astype(_BF)

    vdot = jax.lax.dot_general(vpb, vgb, (((1,), (0,)), ((), ())),
                               preferred_element_type=_F32)
    vpf = vpb.astype(_F32)
    npred = jnp.sum(vpf * vpf, axis=1, keepdims=True)
    vgf = vgb.astype(_F32)
    ntgt = jnp.sum(vgf * vgf, axis=0, keepdims=True)
    viz = jnp.sqrt(jnp.maximum(npred + ntgt - 2.0 * vdot, _EPS))

    base = cost_class + _L_VIS * viz + _L_CTR * center

    acc = base
    group = None
    for j in range(17):
        w05 = pltpu.repeat(w05R_ref[j], rep, 0)
        w4 = pltpu.repeat(w4R_ref[j], rep, 0)
        for k in (0, 1):
            d = 2 * j + k
            zrow = pltpu.repeat(zgR_ref[d], rep, 0)
            u = zpb[:, d:d + 1] - zrow
            a = u + (dcxb if k == 0 else dcyb)
            t = jnp.abs(u) * w05 + jnp.abs(a) * w4
            group = t if group is None else group + t
            if d % _GROUP == _GROUP - 1 or d == 33:
                acc = acc + group.astype(_F32)
                group = None

    out_ref[0] = acc


@jax.jit
def kernel(pred_logits, pred_keypoints, tgt_keypoints, tgt_ids):
    bs, nq, ncls = pred_logits.shape
    T = tgt_keypoints.shape[0]

    tgtT = tgt_keypoints.T
    cgT = tgtT[0:2]
    zgR = jnp.broadcast_to(
        tgtT[2:36].astype(_BF)[:, None, :], (34, 16, T))
    vgb = tgtT[36:53].astype(_BF)
    wabs = jnp.abs(tgtT[36:53])
    w05R = jnp.broadcast_to(
        (_L_DELTAS * wabs).astype(_BF)[:, None, :], (17, 16, T))
    w4R = jnp.broadcast_to(
        (_L_ABS * wabs).astype(_BF)[:, None, :], (17, 16, T))
    ids2d = tgt_ids.reshape(1, T).astype(jnp.int32)

    nblk = nq // _NQ_BLK
    return pl.pallas_call(
        _cost_body,
        grid=(bs, nblk),
        in_specs=[
            pl.BlockSpec((1, _NQ_BLK, ncls), lambda b, q: (b, q, 0)),
            pl.BlockSpec((1, _NQ_BLK, 53), lambda b, q: (b, q, 0)),
            pl.BlockSpec((2, T), lambda b, q: (0, 0)),
            pl.BlockSpec((34, 16, T), lambda b, q: (0, 0, 0)),
            pl.BlockSpec((17, 16, T), lambda b, q: (0, 0, 0)),
            pl.BlockSpec((17, 16, T), lambda b, q: (0, 0, 0)),
            pl.BlockSpec((17, T), lambda b, q: (0, 0)),
            pl.BlockSpec((1, T), lambda b, q: (0, 0)),
        ],
        out_specs=pl.BlockSpec((1, _NQ_BLK, T), lambda b, q: (b, q, 0)),
        out_shape=jax.ShapeDtypeStruct((bs, nq, T), jnp.float32),
        compiler_params=pltpu.CompilerParams(
            dimension_semantics=("parallel", "arbitrary"),
        ),
    )(pred_logits, pred_keypoints, cgT, zgR, w05R, w4R, vgb, ids2d)


# speedup vs baseline: 2.9402x; 1.0616x over previous
import jax
import jax.numpy as jnp
from jax.experimental import pallas as pl
from jax.experimental.pallas import tpu as pltpu

_L_DELTAS = 0.5
_L_VIS = 0.2
_L_CTR = 0.5
_L_ABS = 4.0
_COST_CLASS = 1.0
_EPS = 1e-12
_NQ_BLK = 512
_GROUP = 6

_BF = jnp.bfloat16
_F32 = jnp.float32


def _cost_body(logits_ref, kp_ref, tgtT_ref, ids_ref, out_ref,
               zgR_scr, w05R_scr, w4R_scr, cgT_scr, vgb_scr, is0_scr):
    nq = out_ref.shape[1]
    T = out_ref.shape[2]
    rep = nq // 16

    @pl.when((pl.program_id(0) == 0) & (pl.program_id(1) == 0))
    def _prep():
        tgtT = tgtT_ref[...]
        cgT_scr[...] = tgtT[0:2]
        zgb = tgtT[2:36].astype(_BF)
        vgf = tgtT[36:53]
        vgb_scr[...] = vgf.astype(_BF)
        w05 = (_L_DELTAS * jnp.abs(vgf)).astype(_BF)
        w4 = (_L_ABS * jnp.abs(vgf)).astype(_BF)
        for d in range(34):
            zgR_scr[d] = jnp.broadcast_to(zgb[d:d + 1, :], (16, T))
        for j in range(17):
            w05R_scr[j] = jnp.broadcast_to(w05[j:j + 1, :], (16, T))
            w4R_scr[j] = jnp.broadcast_to(w4[j:j + 1, :], (16, T))
        is0_scr[...] = (ids_ref[...] == 0).astype(_F32)

    logits = logits_ref[0]
    kp = kp_ref[0]
    cp = kp[:, 0:2]
    zpb = kp[:, 2:36].astype(_BF)
    vpb = kp[:, 36:53].astype(_BF)
    cgT = cgT_scr[...]
    vgb = vgb_scr[...]

    l0 = logits[:, 0:1]
    l1 = logits[:, 1:2]
    m = jnp.maximum(l0, l1)
    e0 = jnp.exp(l0 - m)
    e1 = jnp.exp(l1 - m)
    inv = 1.0 / (e0 + e1)
    p0 = e0 * inv
    p1 = e1 * inv
    is0 = is0_scr[...]
    cost_class = -(p1 + (p0 - p1) * is0)

    dcx = cp[:, 0:1] - cgT[0:1, :]
    dcy = cp[:, 1:2] - cgT[1:2, :]
    center = jnp.sqrt(jnp.maximum(dcx * dcx + dcy * dcy, _EPS))
    dcxb = dcx.astype(_BF)
    dcyb = dcy.astype(_BF)

    vdot = jax.lax.dot_general(vpb, vgb, (((1,), (0,)), ((), ())),
                               preferred_element_type=_F32)
    vpf = vpb.astype(_F32)
    npred = jnp.sum(vpf * vpf, axis=1, keepdims=True)
    vgf2 = vgb.astype(_F32)
    ntgt = jnp.sum(vgf2 * vgf2, axis=0, keepdims=True)
    viz = jnp.sqrt(jnp.maximum(npred + ntgt - 2.0 * vdot, _EPS))

    base = cost_class + _L_VIS * viz + _L_CTR * center

    acc = base
    group = None
    for j in range(17):
        w05 = pltpu.repeat(w05R_scr[j], rep, 0)
        w4 = pltpu.repeat(w4R_scr[j], rep, 0)
        for k in (0, 1):
            d = 2 * j + k
            zrow = pltpu.repeat(zgR_scr[d], rep, 0)
            u = zpb[:, d:d + 1] - zrow
            a = u + (dcxb if k == 0 else dcyb)
            t = jnp.abs(u) * w05 + jnp.abs(a) * w4
            group = t if group is None else group + t
            if d % _GROUP == _GROUP - 1 or d == 33:
                acc = acc + group.astype(_F32)
                group = None

    out_ref[0] = acc


@jax.jit
def kernel(pred_logits, pred_keypoints, tgt_keypoints, tgt_ids):
    bs, nq, ncls = pred_logits.shape
    T = tgt_keypoints.shape[0]

    tgtT = tgt_keypoints.T
    ids2d = tgt_ids.reshape(1, T).astype(jnp.int32)

    nblk = nq // _NQ_BLK
    return pl.pallas_call(
        _cost_body,
        grid=(bs, nblk),
        in_specs=[
            pl.BlockSpec((1, _NQ_BLK, ncls), lambda b, q: (b, q, 0)),
            pl.BlockSpec((1, _NQ_BLK, 53), lambda b, q: (b, q, 0)),
            pl.BlockSpec((53, T), lambda b, q: (0, 0)),
            pl.BlockSpec((1, T), lambda b, q: (0, 0)),
        ],
        out_specs=pl.BlockSpec((1, _NQ_BLK, T), lambda b, q: (b, q, 0)),
        out_shape=jax.ShapeDtypeStruct((bs, nq, T), jnp.float32),
        scratch_shapes=[
            pltpu.VMEM((34, 16, T), _BF),
            pltpu.VMEM((17, 16, T), _BF),
            pltpu.VMEM((17, 16, T), _BF),
            pltpu.VMEM((2, T), _F32),
            pltpu.VMEM((17, T), _BF),
            pltpu.VMEM((1, T), _F32),
        ],
        compiler_params=pltpu.CompilerParams(
            dimension_semantics=("parallel", "arbitrary"),
        ),
    )(pred_logits, pred_keypoints, tgtT, ids2d)
